# baseline (speedup 1.0000x reference)
import jax
import jax.numpy as jnp
from jax.experimental import pallas as pl

D_MODEL_K = 2048
N_EXP = 64
BLOCK_T = 512


def _router_body(x_ref, w_ref, idx_ref, p1_ref, prob_ref):
    s = jax.lax.dot_general(
        x_ref[...], w_ref[...], (((1,), (1,)), ((), ())),
        preferred_element_type=jnp.float32)
    m = jnp.max(s, axis=1, keepdims=True)
    e = jnp.exp(s - m)
    denom = jnp.sum(e, axis=1, keepdims=True)
    prob_ref[...] = e / denom
    ii = jax.lax.broadcasted_iota(jnp.int32, s.shape, 1)
    idx_ref[...] = jnp.min(jnp.where(s == m, ii, N_EXP), axis=1)
    p1_ref[...] = 1.0 / denom[:, 0]


def kernel(x, W):
    n_tok = x.shape[0]
    grid = (n_tok // BLOCK_T,)
    out_shapes = (
        jax.ShapeDtypeStruct((n_tok,), jnp.int32),
        jax.ShapeDtypeStruct((n_tok,), jnp.float32),
        jax.ShapeDtypeStruct((n_tok, N_EXP), jnp.float32),
    )
    idx, p1, prob = pl.pallas_call(
        _router_body,
        grid=grid,
        in_specs=[
            pl.BlockSpec((BLOCK_T, D_MODEL_K), lambda i: (i, 0)),
            pl.BlockSpec((N_EXP, D_MODEL_K), lambda i: (0, 0)),
        ],
        out_specs=(
            pl.BlockSpec((BLOCK_T,), lambda i: (i,)),
            pl.BlockSpec((BLOCK_T,), lambda i: (i,)),
            pl.BlockSpec((BLOCK_T, N_EXP), lambda i: (i, 0)),
        ),
        out_shape=out_shapes,
    )(x, W)
    return (idx, p1, prob)


# speedup vs baseline: 1.2682x; 1.2682x over previous
import jax
import jax.numpy as jnp
from jax.experimental import pallas as pl

D_MODEL_K = 2048
N_EXP = 64
BLOCK_T = 512


def _router_body(x_ref, w_ref, idx_ref, p1_ref, prob_ref):
    st = jax.lax.dot_general(
        w_ref[...], x_ref[...], (((1,), (1,)), ((), ())),
        preferred_element_type=jnp.float32)
    m = jnp.max(st, axis=0, keepdims=True)
    e = jnp.exp(st - m)
    denom = jnp.sum(e, axis=0, keepdims=True)
    r = 1.0 / denom
    prob_ref[...] = (e * r).T
    ii = jax.lax.broadcasted_iota(jnp.int32, st.shape, 0)
    idx_ref[0] = jnp.min(jnp.where(st == m, ii, N_EXP), axis=0, keepdims=True)
    p1_ref[0] = r


def kernel(x, W):
    n_tok = x.shape[0]
    g = n_tok // BLOCK_T
    out_shapes = (
        jax.ShapeDtypeStruct((g, 1, BLOCK_T), jnp.int32),
        jax.ShapeDtypeStruct((g, 1, BLOCK_T), jnp.float32),
        jax.ShapeDtypeStruct((n_tok, N_EXP), jnp.float32),
    )
    idx, p1, prob = pl.pallas_call(
        _router_body,
        grid=(g,),
        in_specs=[
            pl.BlockSpec((BLOCK_T, D_MODEL_K), lambda i: (i, 0)),
            pl.BlockSpec((N_EXP, D_MODEL_K), lambda i: (0, 0)),
        ],
        out_specs=(
            pl.BlockSpec((1, 1, BLOCK_T), lambda i: (i, 0, 0)),
            pl.BlockSpec((1, 1, BLOCK_T), lambda i: (i, 0, 0)),
            pl.BlockSpec((BLOCK_T, N_EXP), lambda i: (i, 0)),
        ),
        out_shape=out_shapes,
    )(x, W)
    return (idx.reshape(n_tok), p1.reshape(n_tok), prob)


# speedup vs baseline: 1.4909x; 1.1756x over previous
import jax
import jax.numpy as jnp
from jax.experimental import pallas as pl

D_MODEL_K = 2048
N_EXP = 64
BLOCK_T = 1024


def _router_body(x_ref, w_ref, idx_ref, p1_ref, prob_ref):
    st = jax.lax.dot_general(
        w_ref[...], x_ref[...], (((1,), (1,)), ((), ())),
        preferred_element_type=jnp.float32)
    m = jnp.max(st, axis=0, keepdims=True)
    e = jnp.exp(st - m)
    denom = jnp.sum(e, axis=0, keepdims=True)
    r = 1.0 / denom
    prob_ref[...] = (e * r).T
    ii = jax.lax.broadcasted_iota(jnp.int32, st.shape, 0)
    idx_ref[0] = jnp.min(jnp.where(st == m, ii, N_EXP), axis=0, keepdims=True)
    p1_ref[0] = r


def kernel(x, W):
    n_tok = x.shape[0]
    g = n_tok // BLOCK_T
    out_shapes = (
        jax.ShapeDtypeStruct((g, 1, BLOCK_T), jnp.int32),
        jax.ShapeDtypeStruct((g, 1, BLOCK_T), jnp.float32),
        jax.ShapeDtypeStruct((n_tok, N_EXP), jnp.float32),
    )
    idx, p1, prob = pl.pallas_call(
        _router_body,
        grid=(g,),
        in_specs=[
            pl.BlockSpec((BLOCK_T, D_MODEL_K), lambda i: (i, 0)),
            pl.BlockSpec((N_EXP, D_MODEL_K), lambda i: (0, 0)),
        ],
        out_specs=(
            pl.BlockSpec((1, 1, BLOCK_T), lambda i: (i, 0, 0)),
            pl.BlockSpec((1, 1, BLOCK_T), lambda i: (i, 0, 0)),
            pl.BlockSpec((BLOCK_T, N_EXP), lambda i: (i, 0)),
        ),
        out_shape=out_shapes,
    )(x, W)
    return (idx.reshape(n_tok), p1.reshape(n_tok), prob)


# speedup vs baseline: 1.5020x; 1.0075x over previous
import jax
import jax.numpy as jnp
from jax.experimental import pallas as pl

D_MODEL_K = 2048
N_EXP = 64
BLOCK_T = 2048


def _router_body(x_ref, w_ref, idx_ref, p1_ref, prob_ref):
    st = jax.lax.dot_general(
        w_ref[...], x_ref[...], (((1,), (1,)), ((), ())),
        preferred_element_type=jnp.float32)
    m = jnp.max(st, axis=0, keepdims=True)
    e = jnp.exp(st - m)
    denom = jnp.sum(e, axis=0, keepdims=True)
    r = 1.0 / denom
    prob_ref[...] = (e * r).T
    ii = jax.lax.broadcasted_iota(jnp.int32, st.shape, 0)
    idx_ref[0] = jnp.min(jnp.where(st == m, ii, N_EXP), axis=0, keepdims=True)
    p1_ref[0] = r


def kernel(x, W):
    n_tok = x.shape[0]
    g = n_tok // BLOCK_T
    out_shapes = (
        jax.ShapeDtypeStruct((g, 1, BLOCK_T), jnp.int32),
        jax.ShapeDtypeStruct((g, 1, BLOCK_T), jnp.float32),
        jax.ShapeDtypeStruct((n_tok, N_EXP), jnp.float32),
    )
    idx, p1, prob = pl.pallas_call(
        _router_body,
        grid=(g,),
        in_specs=[
            pl.BlockSpec((BLOCK_T, D_MODEL_K), lambda i: (i, 0)),
            pl.BlockSpec((N_EXP, D_MODEL_K), lambda i: (0, 0)),
        ],
        out_specs=(
            pl.BlockSpec((1, 1, BLOCK_T), lambda i: (i, 0, 0)),
            pl.BlockSpec((1, 1, BLOCK_T), lambda i: (i, 0, 0)),
            pl.BlockSpec((BLOCK_T, N_EXP), lambda i: (i, 0)),
        ),
        out_shape=out_shapes,
    )(x, W)
    return (idx.reshape(n_tok), p1.reshape(n_tok), prob)
